# baseline (speedup 1.0000x reference)
import functools
import numpy as np
import jax
import jax.numpy as jnp
from jax.experimental import pallas as pl
from jax.experimental.pallas import tpu as pltpu

_N = 10000
_C = 128
_DEG = 32
_DELTA = float(np.log(_DEG + 1.0))

_ROWS = 400


def _dense_body(do_relu, x_ref, s_ref, q_ref, mx_ref, mn_ref, deg_ref,
                w_ref, b_ref, o_ref):
    deg = deg_ref[...]
    degc = jnp.maximum(deg, 1.0)
    inv = 1.0 / degc
    s = s_ref[...]
    mean = s * inv
    var = jnp.maximum(q_ref[...] * inv - mean * mean, 0.0)
    std = jnp.sqrt(var + 1e-5)
    has = deg > 0.0
    mx = jnp.where(has, mx_ref[...], 0.0)
    mn = jnp.where(has, mn_ref[...], 0.0)
    logd = jnp.log(deg + 1.0)
    amp = logd * (1.0 / _DELTA)
    att = _DELTA / jnp.clip(logd, 1e-5, None)

    agg = jnp.concatenate([mean, mn, mx, std], axis=1)
    w = w_ref[...]
    out = jnp.dot(x_ref[...], w[0:_C], preferred_element_type=jnp.float32)
    out += jnp.dot(agg, w[_C:5 * _C], preferred_element_type=jnp.float32)
    out += amp * jnp.dot(agg, w[5 * _C:9 * _C], preferred_element_type=jnp.float32)
    out += att * jnp.dot(agg, w[9 * _C:13 * _C], preferred_element_type=jnp.float32)
    out += b_ref[...]
    if do_relu:
        out = jnp.maximum(out, 0.0)
    o_ref[...] = out


def _dense_stage(x, s, q, mx, mn, degf, W, b, do_relu):
    grid = _N // _ROWS
    row_spec = pl.BlockSpec((_ROWS, _C), lambda i: (i, 0))
    out = pl.pallas_call(
        functools.partial(_dense_body, do_relu),
        grid=(grid,),
        in_specs=[
            row_spec, row_spec, row_spec, row_spec, row_spec,
            pl.BlockSpec((_ROWS, 1), lambda i: (i, 0)),
            pl.BlockSpec((13 * _C, _C), lambda i: (0, 0)),
            pl.BlockSpec((1, _C), lambda i: (0, 0)),
        ],
        out_specs=row_spec,
        out_shape=jax.ShapeDtypeStruct((_N, _C), jnp.float32),
    )(x, s, q, mx, mn, degf, W, b)
    return out


def _aggregate(x, src, dst):
    m = x[src]
    s = jax.ops.segment_sum(m, dst, num_segments=_N)
    q = jax.ops.segment_sum(m * m, dst, num_segments=_N)
    mx = jax.ops.segment_max(m, dst, num_segments=_N)
    mn = -jax.ops.segment_max(-m, dst, num_segments=_N)
    return s, q, mx, mn


def kernel(x, edge_index, W0, b0, W1, b1, W2, b2):
    src = edge_index[0]
    dst = edge_index[1]
    ones = jnp.ones((src.shape[0],), dtype=jnp.float32)
    deg = jax.ops.segment_sum(ones, dst, num_segments=_N)
    degf = deg.reshape(_N, 1)

    h = x
    for W, b, relu in ((W0, b0, True), (W1, b1, True), (W2, b2, False)):
        s, q, mx, mn = _aggregate(h, src, dst)
        h = _dense_stage(h, s, q, mx, mn, degf, W, b.reshape(1, _C), relu)
    return h


# speedup vs baseline: 2.6354x; 2.6354x over previous
import functools
import numpy as np
import jax
import jax.numpy as jnp
from jax import lax
from jax.experimental import pallas as pl
from jax.experimental.pallas import tpu as pltpu
from jax.experimental.pallas import tpu_sc as plsc

_N = 10000
_E = 320000
_C = 128
_DEG = 32
_DELTA = float(np.log(_DEG + 1.0))

_NC = 2
_NS = 16
_L = 16
_NW = _NC * _NS

_RNG = 313
_RNGP = 320
_GARB = _RNG
_NPAD = _NW * _RNG
_CH = 2000
_STG = _CH + 16
_ECAP = _E + _CH + 256
_K = 128
_BIG = 3.0e38

_ROWS = 400

_mesh = plsc.VectorSubcoreMesh(core_axis_name="c", subcore_axis_name="s")


def _bin_body(src_hbm, dst_hbm, ls_hbm, ld_hbm, cnt_hbm, deg_hbm,
              srcv, dstv, sts, stdl, degv, cntv):
    cid = lax.axis_index("c")
    sid = lax.axis_index("s")
    w = cid * _NS + sid
    base = w * _ECAP

    zeros16f = jnp.zeros((_L,), jnp.float32)
    for j in range(_RNGP // _L):
        degv[pl.ds(j * _L, _L)] = zeros16f

    ones16 = jnp.ones((_L,), jnp.float32)
    lanes = lax.iota(jnp.int32, _L)
    zeros16i = jnp.zeros((_L,), jnp.int32)
    garb16 = jnp.full((_L,), _GARB, jnp.int32)

    def chunk_body(g, total):
        pltpu.sync_copy(src_hbm.at[pl.ds(g * _CH, _CH)], srcv)
        pltpu.sync_copy(dst_hbm.at[pl.ds(g * _CH, _CH)], dstv)

        def vreg_body(j, cnt):
            s = srcv[pl.ds(j * _L, _L)]
            d = dstv[pl.ds(j * _L, _L)]
            b = d // _RNG
            msk = b == w
            dl = d - w * _RNG
            plsc.store_compressed(sts.at[pl.ds(cnt, _L)], s, mask=msk)
            plsc.store_compressed(stdl.at[pl.ds(cnt, _L)], dl, mask=msk)
            plsc.addupdate_scatter(degv, [dl], ones16, mask=msk)
            return cnt + jnp.sum(msk.astype(jnp.int32))

        cnt = lax.fori_loop(0, _CH // _L, vreg_body, jnp.int32(0))
        pad = (8 - (cnt % 8)) % 8
        mskp = lanes < pad
        plsc.store_compressed(sts.at[pl.ds(cnt, _L)], zeros16i, mask=mskp)
        plsc.store_compressed(stdl.at[pl.ds(cnt, _L)], garb16, mask=mskp)
        cnt = cnt + pad
        fo = pl.multiple_of(base + total, 8)
        pltpu.sync_copy(sts, ls_hbm.at[pl.ds(fo, _STG)])
        pltpu.sync_copy(stdl, ld_hbm.at[pl.ds(fo, _STG)])
        return total + cnt

    total = lax.fori_loop(0, _E // _CH, chunk_body, jnp.int32(0))

    for j in range(_K // _L):
        sts[pl.ds(j * _L, _L)] = zeros16i
        stdl[pl.ds(j * _L, _L)] = garb16
    go = pl.multiple_of(base + total, 8)
    pltpu.sync_copy(sts.at[pl.ds(0, _K)], ls_hbm.at[pl.ds(go, _K)])
    pltpu.sync_copy(stdl.at[pl.ds(0, _K)], ld_hbm.at[pl.ds(go, _K)])

    nb = (total + _K - 1) // _K
    cntv[...] = jnp.where(lanes == 0, nb, 0)
    pltpu.sync_copy(cntv, cnt_hbm.at[w])
    pltpu.sync_copy(degv, deg_hbm.at[w])


_bin_edges = functools.partial(
    pl.kernel,
    out_type=[
        jax.ShapeDtypeStruct((_NW * _ECAP,), jnp.int32),
        jax.ShapeDtypeStruct((_NW * _ECAP,), jnp.int32),
        jax.ShapeDtypeStruct((_NW, _L), jnp.int32),
        jax.ShapeDtypeStruct((_NW, _RNGP), jnp.float32),
    ],
    mesh=_mesh,
    compiler_params=pltpu.CompilerParams(needs_layout_passes=False),
    scratch_types=[
        pltpu.VMEM((_CH,), jnp.int32),
        pltpu.VMEM((_CH,), jnp.int32),
        pltpu.VMEM((_STG,), jnp.int32),
        pltpu.VMEM((_STG,), jnp.int32),
        pltpu.VMEM((_RNGP,), jnp.float32),
        pltpu.VMEM((_L,), jnp.int32),
    ],
)(_bin_body)


_H = _C // 2


def _agg_body(x2_hbm, ls_hbm, ld_hbm, cnt_hbm,
              s0_hbm, s1_hbm, q0_hbm, q1_hbm,
              mx0_hbm, mx1_hbm, mn0_hbm, mn1_hbm,
              sidx, gidx, dlb, gbuf, sacc, qacc, mxa, mna, cntv, sem):
    cid = lax.axis_index("c")
    sid = lax.axis_index("s")
    w = cid * _NS + sid
    base = w * _ECAP

    posbig = jnp.full((_L,), _BIG, jnp.float32)
    negbig = jnp.full((_L,), -_BIG, jnp.float32)
    zeros16 = jnp.zeros((_L,), jnp.float32)

    pltpu.sync_copy(cnt_hbm.at[w], cntv)
    nb = cntv[...][0]

    for half, (s_hbm, q_hbm, mxh_hbm, mnh_hbm) in enumerate(
            ((s0_hbm, q0_hbm, mx0_hbm, mn0_hbm),
             (s1_hbm, q1_hbm, mx1_hbm, mn1_hbm))):

        def init_body(j, _):
            o = pl.ds(j * _L, _L)
            sacc[o] = zeros16
            qacc[o] = zeros16
            mxa[o] = negbig
            mna[o] = posbig
            return 0
        lax.fori_loop(0, _RNGP * _H // _L, init_body, 0)

        def batch_body(b, _):
            off = pl.multiple_of(base + b * _K, 8)
            pltpu.sync_copy(ls_hbm.at[pl.ds(off, _K)], sidx)
            pltpu.sync_copy(ld_hbm.at[pl.ds(off, _K)], dlb)
            for j in range(_K // _L):
                o = pl.ds(j * _L, _L)
                gidx[o] = sidx[o] * 2 + half
            pltpu.async_copy(x2_hbm.at[gidx], gbuf, sem).wait()

            def edge16(jj, _2):
                dvec = dlb[pl.ds(jj * _L, _L)]
                for l in range(_L):
                    dl = dvec[l]
                    i = jj * _L + l
                    roff = dl * _H
                    for c in range(_H // _L):
                        co = c * _L
                        v = gbuf[i, pl.ds(co, _L)]
                        o = pl.ds(roff + co, _L)
                        sacc[o] = sacc[o] + v
                        qacc[o] = qacc[o] + v * v
                        mxa[o] = jnp.maximum(mxa[o], v)
                        mna[o] = jnp.minimum(mna[o], v)
                return 0
            lax.fori_loop(0, _K // _L, edge16, 0)
            return 0

        lax.fori_loop(0, nb, batch_body, 0)

        pltpu.sync_copy(sacc, s_hbm.at[w])
        pltpu.sync_copy(qacc, q_hbm.at[w])
        pltpu.sync_copy(mxa, mxh_hbm.at[w])
        pltpu.sync_copy(mna, mnh_hbm.at[w])


_aggregate_sc = functools.partial(
    pl.kernel,
    out_type=[jax.ShapeDtypeStruct((_NW, _RNGP * _H), jnp.float32)
              for _ in range(8)],
    mesh=_mesh,
    compiler_params=pltpu.CompilerParams(needs_layout_passes=False,
                                         use_tc_tiling_on_sc=False),
    scratch_types=[
        pltpu.VMEM((_K,), jnp.int32),
        pltpu.VMEM((_K,), jnp.int32),
        pltpu.VMEM((_K,), jnp.int32),
        pltpu.VMEM((_K, _H), jnp.float32),
        pltpu.VMEM((_RNGP * _H,), jnp.float32),
        pltpu.VMEM((_RNGP * _H,), jnp.float32),
        pltpu.VMEM((_RNGP * _H,), jnp.float32),
        pltpu.VMEM((_RNGP * _H,), jnp.float32),
        pltpu.VMEM((_L,), jnp.int32),
        pltpu.SemaphoreType.DMA,
    ],
)(_agg_body)


def _dense_body(do_relu, x_ref, s_ref, q_ref, mx_ref, mn_ref, deg_ref,
                w_ref, b_ref, o_ref):
    deg = deg_ref[...]
    degc = jnp.maximum(deg, 1.0)
    inv = 1.0 / degc
    s = s_ref[...]
    mean = s * inv
    var = jnp.maximum(q_ref[...] * inv - mean * mean, 0.0)
    std = jnp.sqrt(var + 1e-5)
    has = deg > 0.0
    mx = jnp.where(has, mx_ref[...], 0.0)
    mn = jnp.where(has, mn_ref[...], 0.0)
    logd = jnp.log(deg + 1.0)
    amp = logd * (1.0 / _DELTA)
    att = _DELTA / jnp.clip(logd, 1e-5, None)

    agg = jnp.concatenate([mean, mn, mx, std], axis=1)
    w = w_ref[...]
    out = jnp.dot(x_ref[...], w[0:_C], preferred_element_type=jnp.float32)
    out += jnp.dot(agg, w[_C:5 * _C], preferred_element_type=jnp.float32)
    out += amp * jnp.dot(agg, w[5 * _C:9 * _C], preferred_element_type=jnp.float32)
    out += att * jnp.dot(agg, w[9 * _C:13 * _C], preferred_element_type=jnp.float32)
    out += b_ref[...]
    if do_relu:
        out = jnp.maximum(out, 0.0)
    o_ref[...] = out


def _dense_stage(x, s, q, mx, mn, degf, W, b, do_relu):
    grid = _N // _ROWS
    row_spec = pl.BlockSpec((_ROWS, _C), lambda i: (i, 0))
    out = pl.pallas_call(
        functools.partial(_dense_body, do_relu),
        grid=(grid,),
        in_specs=[
            row_spec, row_spec, row_spec, row_spec, row_spec,
            pl.BlockSpec((_ROWS, 1), lambda i: (i, 0)),
            pl.BlockSpec((13 * _C, _C), lambda i: (0, 0)),
            pl.BlockSpec((1, _C), lambda i: (0, 0)),
        ],
        out_specs=row_spec,
        out_shape=jax.ShapeDtypeStruct((_N, _C), jnp.float32),
    )(x, s, q, mx, mn, degf, W, b)
    return out


def kernel(x, edge_index, W0, b0, W1, b1, W2, b2):
    src = edge_index[0]
    dst = edge_index[1]

    ls, ld, cnts, deg_rows = _bin_edges(src, dst)
    deg = deg_rows[:, :_RNG].reshape(_NPAD)[:_N]
    degf = deg.reshape(_N, 1)

    def unpad(h0, h1):
        a = jnp.concatenate(
            [h0.reshape(_NW, _RNGP, _H), h1.reshape(_NW, _RNGP, _H)], axis=2)
        return a[:, :_RNG].reshape(_NPAD, _C)[:_N]

    h = x
    for W, b, relu in ((W0, b0, True), (W1, b1, True), (W2, b2, False)):
        x2 = h.reshape(2 * _N, _H)
        s0, s1, q0, q1, mx0, mx1, mn0, mn1 = _aggregate_sc(x2, ls, ld, cnts)
        h = _dense_stage(h, unpad(s0, s1), unpad(q0, q1), unpad(mx0, mx1),
                         unpad(mn0, mn1), degf, W, b.reshape(1, _C), relu)
    return h


# speedup vs baseline: 3.0351x; 1.1517x over previous
import functools
import numpy as np
import jax
import jax.numpy as jnp
from jax import lax
from jax.experimental import pallas as pl
from jax.experimental.pallas import tpu as pltpu
from jax.experimental.pallas import tpu_sc as plsc

_N = 10000
_E = 320000
_C = 128
_DEG = 32
_DELTA = float(np.log(_DEG + 1.0))

_NC = 2
_NS = 16
_L = 16
_NW = _NC * _NS

_NB = 64
_BRNG = 157
_BRP = 160
_GARB = _BRNG
_NPAD = _NB * _BRNG
_CH = 8000
_STG = _CH + 16
_K = 128
_IB = 4096
_ECAP = _E + 16384
_BIG = 3.0e38

_ROWS = 400

_mesh = plsc.VectorSubcoreMesh(core_axis_name="c", subcore_axis_name="s")
_params = pltpu.CompilerParams(needs_layout_passes=False,
                               use_tc_tiling_on_sc=False)


def _bin_body(src_hbm, dst_hbm, ls_hbm, ld_hbm, cnt_hbm, deg_hbm,
              srcv, dstv, st0s, st0d, st1s, st1d, deg0, deg1, cntv):
    cid = lax.axis_index("c")
    sid = lax.axis_index("s")
    w = cid * _NS + sid
    q0 = 2 * w
    q1 = 2 * w + 1

    zeros16f = jnp.zeros((_L,), jnp.float32)
    for j in range(_BRP // _L):
        deg0[pl.ds(j * _L, _L)] = zeros16f
        deg1[pl.ds(j * _L, _L)] = zeros16f

    ones16 = jnp.ones((_L,), jnp.float32)
    lanes = lax.iota(jnp.int32, _L)
    zeros16i = jnp.zeros((_L,), jnp.int32)
    garb16 = jnp.full((_L,), _GARB, jnp.int32)

    def chunk_body(g, tots):
        tot0, tot1 = tots
        pltpu.sync_copy(src_hbm.at[pl.ds(g * _CH, _CH)], srcv)
        pltpu.sync_copy(dst_hbm.at[pl.ds(g * _CH, _CH)], dstv)

        def vreg_body(j, cnts):
            c0, c1 = cnts
            s = srcv[pl.ds(j * _L, _L)]
            d = dstv[pl.ds(j * _L, _L)]
            b = d // _BRNG
            dl = d - b * _BRNG
            m0 = b == q0
            m1 = b == q1
            plsc.store_compressed(st0s.at[pl.ds(c0, _L)], s, mask=m0)
            plsc.store_compressed(st0d.at[pl.ds(c0, _L)], dl, mask=m0)
            plsc.addupdate_scatter(deg0, [dl], ones16, mask=m0)
            plsc.store_compressed(st1s.at[pl.ds(c1, _L)], s, mask=m1)
            plsc.store_compressed(st1d.at[pl.ds(c1, _L)], dl, mask=m1)
            plsc.addupdate_scatter(deg1, [dl], ones16, mask=m1)
            return (c0 + jnp.sum(m0.astype(jnp.int32)),
                    c1 + jnp.sum(m1.astype(jnp.int32)))

        c0, c1 = lax.fori_loop(0, _CH // _L, vreg_body,
                               (jnp.int32(0), jnp.int32(0)), unroll=2)
        p0 = (8 - (c0 % 8)) % 8
        mp0 = lanes < p0
        plsc.store_compressed(st0s.at[pl.ds(c0, _L)], zeros16i, mask=mp0)
        plsc.store_compressed(st0d.at[pl.ds(c0, _L)], garb16, mask=mp0)
        c0 = c0 + p0
        p1 = (8 - (c1 % 8)) % 8
        mp1 = lanes < p1
        plsc.store_compressed(st1s.at[pl.ds(c1, _L)], zeros16i, mask=mp1)
        plsc.store_compressed(st1d.at[pl.ds(c1, _L)], garb16, mask=mp1)
        c1 = c1 + p1
        f0 = pl.multiple_of(q0 * _ECAP + tot0, 8)
        pltpu.sync_copy(st0s, ls_hbm.at[pl.ds(f0, _STG)])
        pltpu.sync_copy(st0d, ld_hbm.at[pl.ds(f0, _STG)])
        f1 = pl.multiple_of(q1 * _ECAP + tot1, 8)
        pltpu.sync_copy(st1s, ls_hbm.at[pl.ds(f1, _STG)])
        pltpu.sync_copy(st1d, ld_hbm.at[pl.ds(f1, _STG)])
        return (tot0 + c0, tot1 + c1)

    tot0, tot1 = lax.fori_loop(0, _E // _CH, chunk_body,
                               (jnp.int32(0), jnp.int32(0)))

    for j in range(2 * _K // _L):
        st0s[pl.ds(j * _L, _L)] = zeros16i
        st0d[pl.ds(j * _L, _L)] = garb16
    g0 = pl.multiple_of(q0 * _ECAP + tot0, 8)
    pltpu.sync_copy(st0s.at[pl.ds(0, 2 * _K)], ls_hbm.at[pl.ds(g0, 2 * _K)])
    pltpu.sync_copy(st0d.at[pl.ds(0, 2 * _K)], ld_hbm.at[pl.ds(g0, 2 * _K)])
    g1 = pl.multiple_of(q1 * _ECAP + tot1, 8)
    pltpu.sync_copy(st0s.at[pl.ds(0, 2 * _K)], ls_hbm.at[pl.ds(g1, 2 * _K)])
    pltpu.sync_copy(st0d.at[pl.ds(0, 2 * _K)], ld_hbm.at[pl.ds(g1, 2 * _K)])

    nb0 = (tot0 + _K - 1) // _K
    nb0 = nb0 + (nb0 % 2)
    nb1 = (tot1 + _K - 1) // _K
    nb1 = nb1 + (nb1 % 2)
    cntv[...] = jnp.where(lanes == 0, nb0, 0)
    pltpu.sync_copy(cntv, cnt_hbm.at[q0])
    cntv[...] = jnp.where(lanes == 0, nb1, 0)
    pltpu.sync_copy(cntv, cnt_hbm.at[q1])
    pltpu.sync_copy(deg0, deg_hbm.at[q0])
    pltpu.sync_copy(deg1, deg_hbm.at[q1])


_bin_edges = functools.partial(
    pl.kernel,
    out_type=[
        jax.ShapeDtypeStruct((_NB * _ECAP,), jnp.int32),
        jax.ShapeDtypeStruct((_NB * _ECAP,), jnp.int32),
        jax.ShapeDtypeStruct((_NB, _L), jnp.int32),
        jax.ShapeDtypeStruct((_NB, _BRP), jnp.float32),
    ],
    mesh=_mesh,
    compiler_params=_params,
    scratch_types=[
        pltpu.VMEM((_CH,), jnp.int32),
        pltpu.VMEM((_CH,), jnp.int32),
        pltpu.VMEM((_STG,), jnp.int32),
        pltpu.VMEM((_STG,), jnp.int32),
        pltpu.VMEM((_STG,), jnp.int32),
        pltpu.VMEM((_STG,), jnp.int32),
        pltpu.VMEM((_BRP,), jnp.float32),
        pltpu.VMEM((_BRP,), jnp.float32),
        pltpu.VMEM((_L,), jnp.int32),
    ],
)(_bin_body)


def _agg_body(x_hbm, ls_hbm, ld_hbm, cnt_hbm,
              sum_hbm, ssq_hbm, mx_hbm, mn_hbm,
              sblk, dblk, gb0, gb1, sacc, qacc, mxa, mna, cntv, sem0, sem1):
    cid = lax.axis_index("c")
    sid = lax.axis_index("s")
    w = cid * _NS + sid

    posbig = jnp.full((_L,), _BIG, jnp.float32)
    negbig = jnp.full((_L,), -_BIG, jnp.float32)
    zeros16 = jnp.zeros((_L,), jnp.float32)

    def rmw(gbuf, kloc):
        def edge16(jj, _):
            dvec = dblk[pl.ds(kloc * _K + jj * _L, _L)]
            for l in range(_L):
                dl = dvec[l]
                i = jj * _L + l
                roff = dl * _C
                for c in range(_C // _L):
                    co = c * _L
                    v = gbuf[i, pl.ds(co, _L)]
                    o = pl.ds(roff + co, _L)
                    sacc[o] = sacc[o] + v
                    qacc[o] = qacc[o] + v * v
                    mxa[o] = jnp.maximum(mxa[o], v)
                    mna[o] = jnp.minimum(mna[o], v)
            return 0
        lax.fori_loop(0, _K // _L, edge16, 0)

    def bin_loop(sub, _):
        q = 2 * w + sub
        base = q * _ECAP

        def init_body(j, _2):
            o = pl.ds(j * _L, _L)
            sacc[o] = zeros16
            qacc[o] = zeros16
            mxa[o] = negbig
            mna[o] = posbig
            return 0
        lax.fori_loop(0, _BRP * _C // _L, init_body, 0)

        pltpu.sync_copy(cnt_hbm.at[q], cntv)
        nbp = cntv[...][0]

        def block_loop(tI, _2):
            boff = pl.multiple_of(base + tI * _IB, 8)
            pltpu.sync_copy(ls_hbm.at[pl.ds(boff, _IB)], sblk)
            pltpu.sync_copy(ld_hbm.at[pl.ds(boff, _IB)], dblk)
            nrem = jnp.minimum(nbp - tI * (_IB // _K), _IB // _K)

            @pl.when(nrem > 0)
            def _prologue():
                pltpu.async_copy(x_hbm.at[sblk.at[pl.ds(0, _K)]], gb0, sem0)

            def pair_loop(t, _3):
                k0 = 2 * t
                k1 = 2 * t + 1
                pltpu.async_copy(
                    x_hbm.at[sblk.at[pl.ds(k1 * _K, _K)]], gb1, sem1)
                pltpu.make_async_copy(x_hbm.at[sblk.at[pl.ds(0, _K)]],
                                      gb0, sem0).wait()
                rmw(gb0, k0)

                @pl.when(k1 + 1 < nrem)
                def _next():
                    pltpu.async_copy(
                        x_hbm.at[sblk.at[pl.ds((k1 + 1) * _K, _K)]], gb0, sem0)

                pltpu.make_async_copy(x_hbm.at[sblk.at[pl.ds(0, _K)]],
                                      gb1, sem1).wait()
                rmw(gb1, k1)
                return 0

            lax.fori_loop(0, nrem // 2, pair_loop, 0)
            return 0

        lax.fori_loop(0, (nbp + _IB // _K - 1) // (_IB // _K), block_loop, 0)

        pltpu.sync_copy(sacc, sum_hbm.at[q])
        pltpu.sync_copy(qacc, ssq_hbm.at[q])
        pltpu.sync_copy(mxa, mx_hbm.at[q])
        pltpu.sync_copy(mna, mn_hbm.at[q])
        return 0

    lax.fori_loop(0, 2, bin_loop, 0)


_aggregate_sc = functools.partial(
    pl.kernel,
    out_type=[jax.ShapeDtypeStruct((_NB, _BRP * _C), jnp.float32)
              for _ in range(4)],
    mesh=_mesh,
    compiler_params=_params,
    scratch_types=[
        pltpu.VMEM((_IB,), jnp.int32),
        pltpu.VMEM((_IB,), jnp.int32),
        pltpu.VMEM((_K, _C), jnp.float32),
        pltpu.VMEM((_K, _C), jnp.float32),
        pltpu.VMEM((_BRP * _C,), jnp.float32),
        pltpu.VMEM((_BRP * _C,), jnp.float32),
        pltpu.VMEM((_BRP * _C,), jnp.float32),
        pltpu.VMEM((_BRP * _C,), jnp.float32),
        pltpu.VMEM((_L,), jnp.int32),
        pltpu.SemaphoreType.DMA,
        pltpu.SemaphoreType.DMA,
    ],
)(_agg_body)


def _dense_body(do_relu, x_ref, s_ref, q_ref, mx_ref, mn_ref, deg_ref,
                w_ref, b_ref, o_ref):
    deg = deg_ref[...]
    degc = jnp.maximum(deg, 1.0)
    inv = 1.0 / degc
    s = s_ref[...]
    mean = s * inv
    var = jnp.maximum(q_ref[...] * inv - mean * mean, 0.0)
    std = jnp.sqrt(var + 1e-5)
    has = deg > 0.0
    mx = jnp.where(has, mx_ref[...], 0.0)
    mn = jnp.where(has, mn_ref[...], 0.0)
    logd = jnp.log(deg + 1.0)
    amp = logd * (1.0 / _DELTA)
    att = _DELTA / jnp.clip(logd, 1e-5, None)

    agg = jnp.concatenate([mean, mn, mx, std], axis=1)
    w = w_ref[...]
    out = jnp.dot(x_ref[...], w[0:_C], preferred_element_type=jnp.float32)
    out += jnp.dot(agg, w[_C:5 * _C], preferred_element_type=jnp.float32)
    out += amp * jnp.dot(agg, w[5 * _C:9 * _C], preferred_element_type=jnp.float32)
    out += att * jnp.dot(agg, w[9 * _C:13 * _C], preferred_element_type=jnp.float32)
    out += b_ref[...]
    if do_relu:
        out = jnp.maximum(out, 0.0)
    o_ref[...] = out


def _dense_stage(x, s, q, mx, mn, degf, W, b, do_relu):
    grid = _N // _ROWS
    row_spec = pl.BlockSpec((_ROWS, _C), lambda i: (i, 0))
    out = pl.pallas_call(
        functools.partial(_dense_body, do_relu),
        grid=(grid,),
        in_specs=[
            row_spec, row_spec, row_spec, row_spec, row_spec,
            pl.BlockSpec((_ROWS, 1), lambda i: (i, 0)),
            pl.BlockSpec((13 * _C, _C), lambda i: (0, 0)),
            pl.BlockSpec((1, _C), lambda i: (0, 0)),
        ],
        out_specs=row_spec,
        out_shape=jax.ShapeDtypeStruct((_N, _C), jnp.float32),
    )(x, s, q, mx, mn, degf, W, b)
    return out


def kernel(x, edge_index, W0, b0, W1, b1, W2, b2):
    src = edge_index[0]
    dst = edge_index[1]

    ls, ld, cnts, deg_rows = _bin_edges(src, dst)
    deg = deg_rows[:, :_BRNG].reshape(_NPAD)[:_N]
    degf = deg.reshape(_N, 1)

    def unpad(a):
        return a.reshape(_NB, _BRP, _C)[:, :_BRNG].reshape(_NPAD, _C)[:_N]

    h = x
    for W, b, relu in ((W0, b0, True), (W1, b1, True), (W2, b2, False)):
        s, q, mxf, mnf = _aggregate_sc(h, ls, ld, cnts)
        h = _dense_stage(h, unpad(s), unpad(q), unpad(mxf), unpad(mnf),
                         degf, W, b.reshape(1, _C), relu)
    return h
